# Initial kernel scaffold; baseline (speedup 1.0000x reference)
import functools

import jax
import jax.numpy as jnp
from jax import lax
from jax.experimental import pallas as pl
from jax.experimental.pallas import tpu as pltpu
from jax.experimental.pallas import tpu_sc as plsc

H = 128
N_NODES = 10000
N_TRIPLES = 320000

NUM_CORES = 2
NUM_SUBCORES = 16
NW = NUM_CORES * NUM_SUBCORES

K = 128
C = 79
PER_W = C * K
TOTAL_PAD = NW * PER_W

DUMMY = N_NODES
NS_ROWS = N_NODES + 8
AGG_ROWS = 10240


def _sc_body(ns_hbm, rs_hbm, subj_hbm, rel_hbm, obj_hbm, agg_hbm,
             agg_sh, subj_v, rel_v, obj_v, buf_a, buf_b, sem_a, sem_b):
    cid = lax.axis_index("c")
    sid = lax.axis_index("s")
    wid = cid * NUM_SUBCORES + sid

    zeros16 = jnp.zeros((16,), jnp.float32)

    def zero_buf(j, carry):
        for t in range(H // 16):
            buf_a[j, pl.ds(t * 16, 16)] = zeros16
        return carry

    lax.fori_loop(0, K, zero_buf, 0)
    for blk in range(AGG_ROWS // NUM_SUBCORES // K):
        pltpu.sync_copy(buf_a, agg_sh.at[pl.ds(sid * 640 + blk * K, K)])

    pltpu.sync_copy(subj_hbm.at[wid], subj_v)
    pltpu.sync_copy(rel_hbm.at[wid], rel_v)
    pltpu.sync_copy(obj_hbm.at[wid], obj_v)

    plsc.subcore_barrier()

    def chunk(c, carry):
        s_idx = subj_v.at[c]
        r_idx = rel_v.at[c]
        o_idx = obj_v.at[c]
        a1 = pltpu.async_copy(rs_hbm.at[r_idx], buf_a, sem_a)
        b1 = pltpu.async_copy(rs_hbm.at[r_idx], buf_b, sem_b)
        a1.wait()
        a2 = pltpu.async_copy(ns_hbm.at[s_idx], buf_a, sem_a, add=True)
        b1.wait()
        b2 = pltpu.async_copy(ns_hbm.at[o_idx], buf_b, sem_b, add=True)
        a2.wait()
        pltpu.sync_copy(buf_a, agg_sh.at[o_idx], add=True)
        b2.wait()
        pltpu.sync_copy(buf_b, agg_sh.at[s_idx], add=True)
        return carry

    lax.fori_loop(0, C, chunk, 0)

    plsc.subcore_barrier()

    rows = N_NODES // NUM_SUBCORES
    pltpu.sync_copy(agg_sh.at[pl.ds(sid * rows, rows)],
                    agg_hbm.at[cid, pl.ds(sid * rows, rows)])


_sc_partial = functools.partial(
    pl.kernel,
    out_type=jax.ShapeDtypeStruct((NUM_CORES, N_NODES, H), jnp.float32),
    mesh=plsc.VectorSubcoreMesh(core_axis_name="c", subcore_axis_name="s"),
    scratch_types=[
        pltpu.VMEM_SHARED((AGG_ROWS, H), jnp.float32),
        pltpu.VMEM((C, K), jnp.int32),
        pltpu.VMEM((C, K), jnp.int32),
        pltpu.VMEM((C, K), jnp.int32),
        pltpu.VMEM((K, H), jnp.float32),
        pltpu.VMEM((K, H), jnp.float32),
        pltpu.SemaphoreType.DMA,
        pltpu.SemaphoreType.DMA,
    ],
)(_sc_body)


def _tc_body(ns_ref, a0_ref, a1_ref, wl_ref, wr_ref, b_ref, o_ref):
    x = ns_ref[...]
    a = a0_ref[...] + a1_ref[...]
    y = (jnp.dot(x, wl_ref[...], preferred_element_type=jnp.float32)
         + jnp.dot(a, wr_ref[...], preferred_element_type=jnp.float32)
         + b_ref[...])
    o_ref[...] = x + y * jax.nn.sigmoid(y)


def _tc_call(ns, a0, a1, wl_t, wr_t, b2d):
    R = 2000
    return pl.pallas_call(
        _tc_body,
        grid=(N_NODES // R,),
        in_specs=[
            pl.BlockSpec((R, H), lambda i: (i, 0)),
            pl.BlockSpec((R, H), lambda i: (i, 0)),
            pl.BlockSpec((R, H), lambda i: (i, 0)),
            pl.BlockSpec((H, H), lambda i: (0, 0)),
            pl.BlockSpec((H, H), lambda i: (0, 0)),
            pl.BlockSpec((1, H), lambda i: (0, 0)),
        ],
        out_specs=pl.BlockSpec((R, H), lambda i: (i, 0)),
        out_shape=jax.ShapeDtypeStruct((N_NODES, H), jnp.float32),
    )(ns, a0, a1, wl_t, wr_t, b2d)


def kernel(node_states, rel_states, triples, W, b):
    tr = triples.astype(jnp.int32)
    pad = TOTAL_PAD - N_TRIPLES
    subj = jnp.concatenate([tr[:, 0], jnp.full((pad,), DUMMY, jnp.int32)])
    rel = jnp.concatenate([tr[:, 1], jnp.zeros((pad,), jnp.int32)])
    obj = jnp.concatenate([tr[:, 2], jnp.full((pad,), DUMMY, jnp.int32)])
    subj = subj.reshape(NW, C, K)
    rel = rel.reshape(NW, C, K)
    obj = obj.reshape(NW, C, K)

    ns_pad = jnp.concatenate(
        [node_states, jnp.zeros((NS_ROWS - N_NODES, H), jnp.float32)])

    agg = _sc_partial(ns_pad, rel_states, subj, rel, obj)

    wl_t = W[:, :H].T
    wr_t = W[:, H:].T
    return _tc_call(node_states, agg[0], agg[1], wl_t, wr_t, b.reshape(1, H))


# speedup vs baseline: 2.4034x; 2.4034x over previous
import functools

import jax
import jax.numpy as jnp
from jax import lax
from jax.experimental import pallas as pl
from jax.experimental.pallas import tpu as pltpu
from jax.experimental.pallas import tpu_sc as plsc

H = 128
N_NODES = 10000
N_TRIPLES = 320000

NUM_CORES = 2
NUM_SUBCORES = 16
NW = NUM_CORES * NUM_SUBCORES

K = 128
C = 80
G = 8
PER_W = C * K
TOTAL_PAD = NW * PER_W

DUMMY = N_NODES
NS_ROWS = N_NODES + 8
AGG_ROWS = 10240


def _sc_body(ns_hbm, rs_hbm, subj_hbm, rel_hbm, obj_hbm, agg_hbm,
             agg_sh, subj_v, rel_v, obj_v, buf_a, buf_b,
             sem_a, sem_b, sem_sa, sem_sb):
    cid = lax.axis_index("c")
    sid = lax.axis_index("s")
    wid = cid * NUM_SUBCORES + sid

    zeros16 = jnp.zeros((16,), jnp.float32)

    def zero_buf(j, carry):
        for t in range(H // 16):
            buf_a[j, pl.ds(t * 16, 16)] = zeros16
        return carry

    lax.fori_loop(0, K, zero_buf, 0)
    for blk in range(AGG_ROWS // NUM_SUBCORES // K):
        pltpu.sync_copy(buf_a, agg_sh.at[pl.ds(sid * 640 + blk * K, K)])

    plsc.subcore_barrier()

    def group(g, carry):
        pltpu.sync_copy(subj_hbm.at[wid, pl.ds(g * G, G)], subj_v)
        pltpu.sync_copy(rel_hbm.at[wid, pl.ds(g * G, G)], rel_v)
        pltpu.sync_copy(obj_hbm.at[wid, pl.ds(g * G, G)], obj_v)
        sa_prev = sb_prev = None
        for cc in range(G):
            s_idx = subj_v.at[cc]
            r_idx = rel_v.at[cc]
            o_idx = obj_v.at[cc]
            if sa_prev is not None:
                sa_prev.wait()
            a1 = pltpu.async_copy(rs_hbm.at[r_idx], buf_a, sem_a)
            if sb_prev is not None:
                sb_prev.wait()
            b1 = pltpu.async_copy(rs_hbm.at[r_idx], buf_b, sem_b)
            a1.wait()
            a2 = pltpu.async_copy(ns_hbm.at[s_idx], buf_a, sem_a, add=True)
            b1.wait()
            b2 = pltpu.async_copy(ns_hbm.at[o_idx], buf_b, sem_b, add=True)
            a2.wait()
            sa_prev = pltpu.async_copy(buf_a, agg_sh.at[o_idx], sem_sa,
                                       add=True)
            b2.wait()
            sb_prev = pltpu.async_copy(buf_b, agg_sh.at[s_idx], sem_sb,
                                       add=True)
        sa_prev.wait()
        sb_prev.wait()
        return carry

    lax.fori_loop(0, C // G, group, 0)

    plsc.subcore_barrier()

    rows = AGG_ROWS // NUM_SUBCORES
    pltpu.sync_copy(agg_sh.at[pl.ds(sid * rows, rows)],
                    agg_hbm.at[cid, pl.ds(sid * rows, rows)])


_sc_partial = functools.partial(
    pl.kernel,
    out_type=jax.ShapeDtypeStruct((NUM_CORES, AGG_ROWS, H), jnp.float32),
    mesh=plsc.VectorSubcoreMesh(core_axis_name="c", subcore_axis_name="s"),
    scratch_types=[
        pltpu.VMEM_SHARED((AGG_ROWS, H), jnp.float32),
        pltpu.VMEM((G, K), jnp.int32),
        pltpu.VMEM((G, K), jnp.int32),
        pltpu.VMEM((G, K), jnp.int32),
        pltpu.VMEM((K, H), jnp.float32),
        pltpu.VMEM((K, H), jnp.float32),
        pltpu.SemaphoreType.DMA,
        pltpu.SemaphoreType.DMA,
        pltpu.SemaphoreType.DMA,
        pltpu.SemaphoreType.DMA,
    ],
)(_sc_body)


def _tc_body(ns_ref, a0_ref, a1_ref, wl_ref, wr_ref, b_ref, o_ref):
    x = ns_ref[...]
    a = a0_ref[...] + a1_ref[...]
    y = (jnp.dot(x, wl_ref[...], preferred_element_type=jnp.float32)
         + jnp.dot(a, wr_ref[...], preferred_element_type=jnp.float32)
         + b_ref[...])
    o_ref[...] = x + y * jax.nn.sigmoid(y)


def _tc_call(ns, a0, a1, wl_t, wr_t, b2d):
    R = 2000
    return pl.pallas_call(
        _tc_body,
        grid=(N_NODES // R,),
        in_specs=[
            pl.BlockSpec((R, H), lambda i: (i, 0)),
            pl.BlockSpec((R, H), lambda i: (i, 0)),
            pl.BlockSpec((R, H), lambda i: (i, 0)),
            pl.BlockSpec((H, H), lambda i: (0, 0)),
            pl.BlockSpec((H, H), lambda i: (0, 0)),
            pl.BlockSpec((1, H), lambda i: (0, 0)),
        ],
        out_specs=pl.BlockSpec((R, H), lambda i: (i, 0)),
        out_shape=jax.ShapeDtypeStruct((N_NODES, H), jnp.float32),
    )(ns, a0, a1, wl_t, wr_t, b2d)


def kernel(node_states, rel_states, triples, W, b):
    tr = triples.astype(jnp.int32)
    pad = TOTAL_PAD - N_TRIPLES
    subj = jnp.concatenate([tr[:, 0], jnp.full((pad,), DUMMY, jnp.int32)])
    rel = jnp.concatenate([tr[:, 1], jnp.zeros((pad,), jnp.int32)])
    obj = jnp.concatenate([tr[:, 2], jnp.full((pad,), DUMMY, jnp.int32)])
    subj = subj.reshape(NW, C, K)
    rel = rel.reshape(NW, C, K)
    obj = obj.reshape(NW, C, K)

    ns_pad = jnp.concatenate(
        [node_states, jnp.zeros((NS_ROWS - N_NODES, H), jnp.float32)])

    agg = _sc_partial(ns_pad, rel_states, subj, rel, obj)

    wl_t = W[:, :H].T
    wr_t = W[:, H:].T
    return _tc_call(node_states, agg[0, :N_NODES], agg[1, :N_NODES],
                    wl_t, wr_t, b.reshape(1, H))


# speedup vs baseline: 3.6760x; 1.5295x over previous
import functools

import jax
import jax.numpy as jnp
from jax import lax
from jax.experimental import pallas as pl
from jax.experimental.pallas import tpu as pltpu
from jax.experimental.pallas import tpu_sc as plsc

H = 128
N_NODES = 10000
N_TRIPLES = 320000

NUM_CORES = 2
NUM_SUBCORES = 16
NW = NUM_CORES * NUM_SUBCORES

K = 128
C0 = 120
C1 = 40
G = 8
CT = NUM_SUBCORES * (C0 + C1)
TOTAL_PAD = CT * K

DUMMY = N_NODES
NS_ROWS = N_NODES + 8
AGG_ROWS = 10240


def _sc_body(ns_hbm, rs_hbm, subj_hbm, rel_hbm, obj_hbm, agg_hbm,
             agg_sh, subj_v, rel_v, obj_v, buf_a, buf_b,
             sem_a, sem_b, sem_sa, sem_sb):
    cid = lax.axis_index("c")
    sid = lax.axis_index("s")
    wid = cid * NUM_SUBCORES + sid

    zeros16 = jnp.zeros((16,), jnp.float32)

    def zero_buf(j, carry):
        for t in range(H // 16):
            buf_a[j, pl.ds(t * 16, 16)] = zeros16
        return carry

    lax.fori_loop(0, K, zero_buf, 0)
    for blk in range(AGG_ROWS // NUM_SUBCORES // K):
        pltpu.sync_copy(buf_a, agg_sh.at[pl.ds(sid * 640 + blk * K, K)])

    plsc.subcore_barrier()

    start = jnp.where(cid == 0, sid * C0, NUM_SUBCORES * C0 + sid * C1)
    n_groups = jnp.where(cid == 0, C0 // G, C1 // G)

    def group(g, carry):
        base = pl.multiple_of(start + g * G, 8)
        pltpu.sync_copy(subj_hbm.at[pl.ds(base, G)], subj_v)
        pltpu.sync_copy(rel_hbm.at[pl.ds(base, G)], rel_v)
        pltpu.sync_copy(obj_hbm.at[pl.ds(base, G)], obj_v)
        sa_prev = sb_prev = None
        for cc in range(G):
            s_idx = subj_v.at[cc]
            r_idx = rel_v.at[cc]
            o_idx = obj_v.at[cc]
            if sa_prev is not None:
                sa_prev.wait()
            a1 = pltpu.async_copy(rs_hbm.at[r_idx], buf_a, sem_a)
            if sb_prev is not None:
                sb_prev.wait()
            b1 = pltpu.async_copy(rs_hbm.at[r_idx], buf_b, sem_b)
            a1.wait()
            a2 = pltpu.async_copy(ns_hbm.at[s_idx], buf_a, sem_a, add=True)
            b1.wait()
            b2 = pltpu.async_copy(ns_hbm.at[o_idx], buf_b, sem_b, add=True)
            a2.wait()
            sa_prev = pltpu.async_copy(buf_a, agg_sh.at[o_idx], sem_sa,
                                       add=True)
            b2.wait()
            sb_prev = pltpu.async_copy(buf_b, agg_sh.at[s_idx], sem_sb,
                                       add=True)
        sa_prev.wait()
        sb_prev.wait()
        return carry

    lax.fori_loop(0, n_groups, group, 0)

    plsc.subcore_barrier()

    rows = AGG_ROWS // NUM_SUBCORES
    pltpu.sync_copy(agg_sh.at[pl.ds(sid * rows, rows)],
                    agg_hbm.at[cid, pl.ds(sid * rows, rows)])


_sc_partial = functools.partial(
    pl.kernel,
    out_type=jax.ShapeDtypeStruct((NUM_CORES, AGG_ROWS, H), jnp.float32),
    mesh=plsc.VectorSubcoreMesh(core_axis_name="c", subcore_axis_name="s"),
    scratch_types=[
        pltpu.VMEM_SHARED((AGG_ROWS, H), jnp.float32),
        pltpu.VMEM((G, K), jnp.int32),
        pltpu.VMEM((G, K), jnp.int32),
        pltpu.VMEM((G, K), jnp.int32),
        pltpu.VMEM((K, H), jnp.float32),
        pltpu.VMEM((K, H), jnp.float32),
        pltpu.SemaphoreType.DMA,
        pltpu.SemaphoreType.DMA,
        pltpu.SemaphoreType.DMA,
        pltpu.SemaphoreType.DMA,
    ],
)(_sc_body)


def _tc_body(ns_ref, a0_ref, a1_ref, wl_ref, wr_ref, b_ref, o_ref):
    x = ns_ref[...]
    a = a0_ref[...] + a1_ref[...]
    y = (jnp.dot(x, wl_ref[...], preferred_element_type=jnp.float32)
         + jnp.dot(a, wr_ref[...], preferred_element_type=jnp.float32)
         + b_ref[...])
    o_ref[...] = x + y * jax.nn.sigmoid(y)


def _tc_call(ns, a0, a1, wl_t, wr_t, b2d):
    R = 2000
    return pl.pallas_call(
        _tc_body,
        grid=(N_NODES // R,),
        in_specs=[
            pl.BlockSpec((R, H), lambda i: (i, 0)),
            pl.BlockSpec((R, H), lambda i: (i, 0)),
            pl.BlockSpec((R, H), lambda i: (i, 0)),
            pl.BlockSpec((H, H), lambda i: (0, 0)),
            pl.BlockSpec((H, H), lambda i: (0, 0)),
            pl.BlockSpec((1, H), lambda i: (0, 0)),
        ],
        out_specs=pl.BlockSpec((R, H), lambda i: (i, 0)),
        out_shape=jax.ShapeDtypeStruct((N_NODES, H), jnp.float32),
    )(ns, a0, a1, wl_t, wr_t, b2d)


def kernel(node_states, rel_states, triples, W, b):
    tr = triples.astype(jnp.int32)
    pad = TOTAL_PAD - N_TRIPLES
    subj = jnp.concatenate([tr[:, 0], jnp.full((pad,), DUMMY, jnp.int32)])
    rel = jnp.concatenate([tr[:, 1], jnp.zeros((pad,), jnp.int32)])
    obj = jnp.concatenate([tr[:, 2], jnp.full((pad,), DUMMY, jnp.int32)])
    subj = subj.reshape(CT, K)
    rel = rel.reshape(CT, K)
    obj = obj.reshape(CT, K)

    ns_pad = jnp.concatenate(
        [node_states, jnp.zeros((NS_ROWS - N_NODES, H), jnp.float32)])

    agg = _sc_partial(ns_pad, rel_states, subj, rel, obj)

    wl_t = W[:, :H].T
    wr_t = W[:, H:].T
    return _tc_call(node_states, agg[0, :N_NODES], agg[1, :N_NODES],
                    wl_t, wr_t, b.reshape(1, H))


# speedup vs baseline: 4.2960x; 1.1687x over previous
import functools

import jax
import jax.numpy as jnp
from jax import lax
from jax.experimental import pallas as pl
from jax.experimental.pallas import tpu as pltpu
from jax.experimental.pallas import tpu_sc as plsc

H = 128
N_NODES = 10000
N_TRIPLES = 320000

NUM_CORES = 2
NUM_SUBCORES = 16
NW = NUM_CORES * NUM_SUBCORES

K = 128
C0 = 136
C1 = 24
G = 8
CT = NUM_SUBCORES * (C0 + C1)
TOTAL_PAD = CT * K

DUMMY = N_NODES
NS_ROWS = N_NODES + 8
AGG_ROWS = 10240


def _sc_body(ns_hbm, rs_hbm, subj_hbm, rel_hbm, obj_hbm, agg_hbm,
             agg_sh, subj_v, rel_v, obj_v, buf_a, buf_b,
             sem_a, sem_b, sem_sa, sem_sb):
    cid = lax.axis_index("c")
    sid = lax.axis_index("s")
    wid = cid * NUM_SUBCORES + sid

    zeros16 = jnp.zeros((16,), jnp.float32)

    def zero_buf(j, carry):
        for t in range(H // 16):
            buf_a[j, pl.ds(t * 16, 16)] = zeros16
        return carry

    lax.fori_loop(0, K, zero_buf, 0)
    for blk in range(AGG_ROWS // NUM_SUBCORES // K):
        pltpu.sync_copy(buf_a, agg_sh.at[pl.ds(sid * 640 + blk * K, K)])

    plsc.subcore_barrier()

    start = jnp.where(cid == 0, sid * C0, NUM_SUBCORES * C0 + sid * C1)
    n_groups = jnp.where(cid == 0, C0 // G, C1 // G)

    def group(g, carry):
        base = pl.multiple_of(start + g * G, 8)
        pltpu.sync_copy(subj_hbm.at[pl.ds(base, G)], subj_v)
        pltpu.sync_copy(rel_hbm.at[pl.ds(base, G)], rel_v)
        pltpu.sync_copy(obj_hbm.at[pl.ds(base, G)], obj_v)
        sa_prev = sb_prev = None
        for cc in range(G):
            s_idx = subj_v.at[cc]
            r_idx = rel_v.at[cc]
            o_idx = obj_v.at[cc]
            if sa_prev is not None:
                sa_prev.wait()
            a1 = pltpu.async_copy(rs_hbm.at[r_idx], buf_a, sem_a)
            if sb_prev is not None:
                sb_prev.wait()
            b1 = pltpu.async_copy(rs_hbm.at[r_idx], buf_b, sem_b)
            a1.wait()
            a2 = pltpu.async_copy(ns_hbm.at[s_idx], buf_a, sem_a, add=True)
            b1.wait()
            b2 = pltpu.async_copy(ns_hbm.at[o_idx], buf_b, sem_b, add=True)
            a2.wait()
            sa_prev = pltpu.async_copy(buf_a, agg_sh.at[o_idx], sem_sa,
                                       add=True)
            b2.wait()
            sb_prev = pltpu.async_copy(buf_b, agg_sh.at[s_idx], sem_sb,
                                       add=True)
        sa_prev.wait()
        sb_prev.wait()
        return carry

    lax.fori_loop(0, n_groups, group, 0)

    plsc.subcore_barrier()

    rows = AGG_ROWS // NUM_SUBCORES
    pltpu.sync_copy(agg_sh.at[pl.ds(sid * rows, rows)],
                    agg_hbm.at[cid, pl.ds(sid * rows, rows)])


_sc_partial = functools.partial(
    pl.kernel,
    out_type=jax.ShapeDtypeStruct((NUM_CORES, AGG_ROWS, H), jnp.float32),
    mesh=plsc.VectorSubcoreMesh(core_axis_name="c", subcore_axis_name="s"),
    scratch_types=[
        pltpu.VMEM_SHARED((AGG_ROWS, H), jnp.float32),
        pltpu.VMEM((G, K), jnp.int32),
        pltpu.VMEM((G, K), jnp.int32),
        pltpu.VMEM((G, K), jnp.int32),
        pltpu.VMEM((K, H), jnp.float32),
        pltpu.VMEM((K, H), jnp.float32),
        pltpu.SemaphoreType.DMA,
        pltpu.SemaphoreType.DMA,
        pltpu.SemaphoreType.DMA,
        pltpu.SemaphoreType.DMA,
    ],
)(_sc_body)


def _tc_body(ns_ref, a0_ref, a1_ref, wl_ref, wr_ref, b_ref, o_ref):
    x = ns_ref[...]
    a = a0_ref[...] + a1_ref[...]
    y = (jnp.dot(x, wl_ref[...], preferred_element_type=jnp.float32)
         + jnp.dot(a, wr_ref[...], preferred_element_type=jnp.float32)
         + b_ref[...])
    o_ref[...] = x + y * jax.nn.sigmoid(y)


def _tc_call(ns, a0, a1, wl_t, wr_t, b2d):
    R = 2000
    return pl.pallas_call(
        _tc_body,
        grid=(N_NODES // R,),
        in_specs=[
            pl.BlockSpec((R, H), lambda i: (i, 0)),
            pl.BlockSpec((R, H), lambda i: (i, 0)),
            pl.BlockSpec((R, H), lambda i: (i, 0)),
            pl.BlockSpec((H, H), lambda i: (0, 0)),
            pl.BlockSpec((H, H), lambda i: (0, 0)),
            pl.BlockSpec((1, H), lambda i: (0, 0)),
        ],
        out_specs=pl.BlockSpec((R, H), lambda i: (i, 0)),
        out_shape=jax.ShapeDtypeStruct((N_NODES, H), jnp.float32),
    )(ns, a0, a1, wl_t, wr_t, b2d)


def kernel(node_states, rel_states, triples, W, b):
    tr = triples.astype(jnp.int32)
    pad = TOTAL_PAD - N_TRIPLES
    subj = jnp.concatenate([tr[:, 0], jnp.full((pad,), DUMMY, jnp.int32)])
    rel = jnp.concatenate([tr[:, 1], jnp.zeros((pad,), jnp.int32)])
    obj = jnp.concatenate([tr[:, 2], jnp.full((pad,), DUMMY, jnp.int32)])
    subj = subj.reshape(CT, K)
    rel = rel.reshape(CT, K)
    obj = obj.reshape(CT, K)

    ns_pad = jnp.concatenate(
        [node_states, jnp.zeros((NS_ROWS - N_NODES, H), jnp.float32)])

    agg = _sc_partial(ns_pad, rel_states, subj, rel, obj)

    wl_t = W[:, :H].T
    wr_t = W[:, H:].T
    return _tc_call(node_states, agg[0, :N_NODES], agg[1, :N_NODES],
                    wl_t, wr_t, b.reshape(1, H))


# speedup vs baseline: 5.3261x; 1.2398x over previous
import functools

import jax
import jax.numpy as jnp
from jax import lax
from jax.experimental import pallas as pl
from jax.experimental.pallas import tpu as pltpu
from jax.experimental.pallas import tpu_sc as plsc

H = 128
N_NODES = 10000
N_TRIPLES = 320000

NUM_CORES = 2
NUM_SUBCORES = 16

K = 56
C0 = 312
C1 = 48
G = 8
CT = NUM_SUBCORES * (C0 + C1)
TOTAL_PAD = CT * K

DUMMY = N_NODES
NS_ROWS = N_NODES + 8
AGG_ROWS = 10112
ZROWS = AGG_ROWS // NUM_SUBCORES


def _sc_body(ns_hbm, rs_hbm, zeros_hbm, subj_hbm, rel_hbm, obj_hbm, agg_hbm,
             agg_sh, subj_v, rel_v, obj_v,
             ba0, ba1, ba2, bb0, bb1, bb2,
             ga0, ga1, ga2, gb0, gb1, gb2,
             sa0, sa1, sa2, sb0, sb1, sb2):
    cid = lax.axis_index("c")
    sid = lax.axis_index("s")

    bufs_a = [ba0, ba1, ba2]
    bufs_b = [bb0, bb1, bb2]
    gsem_a = [ga0, ga1, ga2]
    gsem_b = [gb0, gb1, gb2]
    ssem_a = [sa0, sa1, sa2]
    ssem_b = [sb0, sb1, sb2]

    pltpu.sync_copy(zeros_hbm, agg_sh.at[pl.ds(sid * ZROWS, ZROWS)])

    plsc.subcore_barrier()

    start = jnp.where(cid == 0, sid * C0, NUM_SUBCORES * C0 + sid * C1)
    n_groups = jnp.where(cid == 0, C0 // G, C1 // G)

    def group(g, carry):
        base = pl.multiple_of(start + g * G, 8)
        pltpu.sync_copy(subj_hbm.at[pl.ds(base, G)], subj_v)
        pltpu.sync_copy(rel_hbm.at[pl.ds(base, G)], rel_v)
        pltpu.sync_copy(obj_hbm.at[pl.ds(base, G)], obj_v)

        gath_a = [None] * G
        gath_b = [None] * G
        gadd_a = [None] * G
        gadd_b = [None] * G
        pend_sa = [None, None, None]
        pend_sb = [None, None, None]

        def stage2(c):
            q = c % 3
            gath_a[c].wait()
            gadd_a[c] = pltpu.async_copy(
                ns_hbm.at[subj_v.at[c]], bufs_a[q], gsem_a[q], add=True)
            gath_b[c].wait()
            gadd_b[c] = pltpu.async_copy(
                ns_hbm.at[obj_v.at[c]], bufs_b[q], gsem_b[q], add=True)

        def stage3(c):
            r = c % 3
            gadd_a[c].wait()
            pend_sa[r] = pltpu.async_copy(
                bufs_a[r], agg_sh.at[obj_v.at[c]], ssem_a[r], add=True)
            gadd_b[c].wait()
            pend_sb[r] = pltpu.async_copy(
                bufs_b[r], agg_sh.at[subj_v.at[c]], ssem_b[r], add=True)

        for cc in range(G):
            p = cc % 3
            if pend_sa[p] is not None:
                pend_sa[p].wait()
            gath_a[cc] = pltpu.async_copy(
                rs_hbm.at[rel_v.at[cc]], bufs_a[p], gsem_a[p])
            if pend_sb[p] is not None:
                pend_sb[p].wait()
            gath_b[cc] = pltpu.async_copy(
                rs_hbm.at[rel_v.at[cc]], bufs_b[p], gsem_b[p])
            if cc >= 1:
                stage2(cc - 1)
            if cc >= 2:
                stage3(cc - 2)

        stage2(G - 1)
        stage3(G - 2)
        stage3(G - 1)
        for p in range(3):
            if pend_sa[p] is not None:
                pend_sa[p].wait()
            if pend_sb[p] is not None:
                pend_sb[p].wait()
        return carry

    lax.fori_loop(0, n_groups, group, 0)

    plsc.subcore_barrier()

    pltpu.sync_copy(agg_sh.at[pl.ds(sid * ZROWS, ZROWS)],
                    agg_hbm.at[cid, pl.ds(sid * ZROWS, ZROWS)])


_sc_partial = functools.partial(
    pl.kernel,
    out_type=jax.ShapeDtypeStruct((NUM_CORES, AGG_ROWS, H), jnp.float32),
    mesh=plsc.VectorSubcoreMesh(core_axis_name="c", subcore_axis_name="s"),
    scratch_types=[
        pltpu.VMEM_SHARED((AGG_ROWS, H), jnp.float32),
        pltpu.VMEM((G, K), jnp.int32),
        pltpu.VMEM((G, K), jnp.int32),
        pltpu.VMEM((G, K), jnp.int32),
        pltpu.VMEM((K, H), jnp.float32),
        pltpu.VMEM((K, H), jnp.float32),
        pltpu.VMEM((K, H), jnp.float32),
        pltpu.VMEM((K, H), jnp.float32),
        pltpu.VMEM((K, H), jnp.float32),
        pltpu.VMEM((K, H), jnp.float32),
    ] + [pltpu.SemaphoreType.DMA] * 12,
)(_sc_body)


def _tc_body(ns_ref, a0_ref, a1_ref, wl_ref, wr_ref, b_ref, o_ref):
    x = ns_ref[...]
    a = a0_ref[...] + a1_ref[...]
    y = (jnp.dot(x, wl_ref[...], preferred_element_type=jnp.float32)
         + jnp.dot(a, wr_ref[...], preferred_element_type=jnp.float32)
         + b_ref[...])
    o_ref[...] = x + y * jax.nn.sigmoid(y)


def _tc_call(ns, a0, a1, wl_t, wr_t, b2d):
    R = 2000
    return pl.pallas_call(
        _tc_body,
        grid=(N_NODES // R,),
        in_specs=[
            pl.BlockSpec((R, H), lambda i: (i, 0)),
            pl.BlockSpec((R, H), lambda i: (i, 0)),
            pl.BlockSpec((R, H), lambda i: (i, 0)),
            pl.BlockSpec((H, H), lambda i: (0, 0)),
            pl.BlockSpec((H, H), lambda i: (0, 0)),
            pl.BlockSpec((1, H), lambda i: (0, 0)),
        ],
        out_specs=pl.BlockSpec((R, H), lambda i: (i, 0)),
        out_shape=jax.ShapeDtypeStruct((N_NODES, H), jnp.float32),
    )(ns, a0, a1, wl_t, wr_t, b2d)


def kernel(node_states, rel_states, triples, W, b):
    tr = triples.astype(jnp.int32)
    pad = TOTAL_PAD - N_TRIPLES
    subj = jnp.concatenate([tr[:, 0], jnp.full((pad,), DUMMY, jnp.int32)])
    rel = jnp.concatenate([tr[:, 1], jnp.zeros((pad,), jnp.int32)])
    obj = jnp.concatenate([tr[:, 2], jnp.full((pad,), DUMMY, jnp.int32)])
    subj = subj.reshape(CT, K)
    rel = rel.reshape(CT, K)
    obj = obj.reshape(CT, K)

    ns_pad = jnp.concatenate(
        [node_states, jnp.zeros((NS_ROWS - N_NODES, H), jnp.float32)])
    zeros = jnp.zeros((ZROWS, H), jnp.float32)

    agg = _sc_partial(ns_pad, rel_states, zeros, subj, rel, obj)

    wl_t = W[:, :H].T
    wr_t = W[:, H:].T
    return _tc_call(node_states, agg[0, :N_NODES], agg[1, :N_NODES],
                    wl_t, wr_t, b.reshape(1, H))
